# baseline (speedup 1.0000x reference)
import jax
import jax.numpy as jnp
from jax.experimental import pallas as pl
from jax.experimental.pallas import tpu as pltpu

_CH = 8
_HID = 32
_KS = 5
_NL = 4
_NF = 4
_HC = _CH // 2
_PAD = (_KS - 1) // 2
_BB = 8


def _flows_kernel(x_ref, m_ref, in_ref, pre_ref, rs_ref, skip_ref,
                  post_ref, ind1_ref, out_ref):
    T = x_ref.shape[-1]
    f32, bf16 = jnp.float32, jnp.bfloat16
    H = _HID
    half = jnp.bfloat16(0.5)

    tpos = jax.lax.broadcasted_iota(jnp.int32, (1, T), 1)
    edge = {}
    for d in range(-_PAD, _PAD + 1):
        if d < 0:
            edge[d] = (tpos >= -d).astype(bf16)
        elif d > 0:
            edge[d] = (tpos < T - d).astype(bf16)

    for b in range(_BB):
        s = x_ref[b]
        mask = m_ref[b]
        mq = mask.astype(bf16)
        em = {0: mq}
        for d in range(-_PAD, _PAD + 1):
            if d != 0:
                rotm = jnp.concatenate([mq[:, d:], mq[:, :d]], axis=1)
                em[d] = rotm * edge[d]
        for f in range(_NF):
            sx = jnp.concatenate([s.astype(bf16), mq, mq], axis=0)
            xcur = jnp.dot(pre_ref[f], sx, preferred_element_type=f32)
            skip = None
            for i in range(_NL):
                xq = xcur.astype(bf16)
                taps = []
                for j in range(_KS):
                    d = j - _PAD
                    if d == 0:
                        taps.append(xq * em[0])
                    else:
                        rot = jnp.concatenate([xq[:, d:], xq[:, :d]], axis=1)
                        taps.append(rot * em[d])
                tx = jnp.concatenate(taps + [mq, mq], axis=0)
                z = jnp.dot(in_ref[b, f * _NL + i], tx,
                            preferred_element_type=f32)
                tz = jnp.tanh(z).astype(bf16)
                acts = tz[:H] * (tz[H:] * half + half)
                ax = jnp.concatenate([acts, mq, mq], axis=0)
                if i < _NL - 1:
                    rs = jnp.dot(rs_ref[f, i], ax,
                                 preferred_element_type=f32)
                    xcur = xcur + rs[:H]
                    sk = rs[H:]
                else:
                    sk = jnp.dot(skip_ref[f], ax,
                                 preferred_element_type=f32)
                skip = sk if skip is None else skip + sk
            kx = jnp.concatenate([skip.astype(bf16), mq, mq], axis=0)
            mf = jnp.dot(post_ref[f], kx,
                         preferred_element_type=f32) * mask
            blend = 1.0 + ind1_ref[f] * (mask - 1.0)
            s = s * blend + mf
        out_ref[b] = s.astype(out_ref.dtype)


def _hi_lo(v):
    hi = v.astype(jnp.bfloat16)
    lo = (v - hi.astype(jnp.float32)).astype(jnp.bfloat16)
    return hi, lo


def _fold_bias(w, b):
    hi, lo = _hi_lo(b)
    return jnp.concatenate([w, hi, lo], axis=-1)


def kernel(x, x_mask, g, pre_w, pre_b, in_w, rs_w, rs_b, skip_w, skip_b,
           post_w, post_b, ind1, cond_w, cond_b, in_b, gate_scale):
    B, C, T = x.shape
    FL = _NF * _NL
    H2 = 2 * _HID

    g2 = g[:, :, 0]
    ga = jnp.einsum('bg,fog->fbo', g2, cond_w) + cond_b[:, None]
    ga = ga.reshape(_NF, B, _NL, H2) + in_b[:, None]
    gb = jnp.transpose(ga, (1, 0, 2, 3)).reshape(B, FL, H2)
    gb = gb * gate_scale

    ghi, glo = _hi_lo(gb[..., None])
    in_ext = jnp.concatenate(
        [jnp.broadcast_to(in_w.reshape(1, FL, H2, _KS * _HID),
                          (B, FL, H2, _KS * _HID)), ghi, glo], axis=-1)
    pre_ext = _fold_bias(pre_w, pre_b)
    rs_ext = _fold_bias(rs_w, rs_b)
    skip_ext = _fold_bias(skip_w, skip_b)
    post_ext = _fold_bias(post_w, post_b)

    weights = [pre_ext, rs_ext, skip_ext, post_ext, ind1]
    full = lambda a: pl.BlockSpec(a.shape, (lambda nd: (lambda p: (0,) * nd))(a.ndim))

    y = pl.pallas_call(
        _flows_kernel,
        out_shape=jax.ShapeDtypeStruct((B, C, T), x.dtype),
        grid=(B // _BB,),
        in_specs=[
            pl.BlockSpec((_BB, C, T), lambda p: (p, 0, 0)),
            pl.BlockSpec((_BB, 1, T), lambda p: (p, 0, 0)),
            pl.BlockSpec((_BB, FL, H2, _KS * _HID + 2),
                         lambda p: (p, 0, 0, 0)),
        ] + [full(w) for w in weights],
        out_specs=pl.BlockSpec((_BB, C, T), lambda p: (p, 0, 0)),
        compiler_params=pltpu.CompilerParams(
            dimension_semantics=("parallel",)),
    )(x, x_mask, in_ext, *weights)
    return y
